# baseline (speedup 1.0000x reference)
import functools

import jax
import jax.numpy as jnp
from jax import lax
from jax.experimental import pallas as pl
from jax.experimental.pallas import tpu as pltpu
from jax.experimental.pallas import tpu_sc as plsc

_S = 7
_INF = float("inf")
_IP = False


def _bf(x):
    return x.astype(jnp.bfloat16).astype(jnp.float32)


def _knn_body(vt_ref, v_ref, ndn_ref, nidx_ref, *, V, RB):
    b = pl.program_id(0)
    vt = [vt_ref[0, d:d + 1, :] for d in range(3)]
    qc = vt[0] * vt[0] + vt[1] * vt[1] + vt[2] * vt[2]
    vb = [v_ref[0, :, d:d + 1] for d in range(3)]
    vtb = [_bf(x) for x in vt]
    vbb = [_bf(x) for x in vb]
    inner = (vbb[0] * vtb[0] + vbb[1] * vtb[1]) + vbb[2] * vtb[2]
    qr = vb[0] * vb[0] + vb[1] * vb[1] + vb[2] * vb[2]
    cur = (-2.0 * inner + qc) + qr
    colI = lax.broadcasted_iota(jnp.int32, (RB, V), 1)
    for n in range(4):
        m = jnp.min(cur, axis=1, keepdims=True)
        a = jnp.min(jnp.where(cur == m, colI, jnp.int32(2 ** 30)),
                    axis=1, keepdims=True)
        oh = colI == a
        cur = jnp.where(oh, _INF, cur)
        if n == 0:
            continue
        nb = [jnp.sum(jnp.where(oh, vt[d], 0.0), axis=1, keepdims=True)
              for d in range(3)]
        dirs = [nb[d] - vb[d] for d in range(3)]
        nrm = jnp.sqrt(dirs[0] * dirs[0] + dirs[1] * dirs[1]
                       + dirs[2] * dirs[2])
        den = jnp.maximum(nrm, 1e-12)
        j = n - 1
        for d in range(3):
            ndn_ref[0, :, 3 * j + d:3 * j + d + 1] = dirs[d] / den
        nidx_ref[0, :, j:j + 1] = a + b * V


def _knn(vertices, V, RB=512):
    vt = jnp.transpose(vertices, (0, 2, 1))
    grid = (4, V // RB)
    ndn, nidx = pl.pallas_call(
        functools.partial(_knn_body, V=V, RB=RB),
        grid=grid,
        in_specs=[
            pl.BlockSpec((1, 3, V), lambda b, r: (b, 0, 0)),
            pl.BlockSpec((1, RB, 3), lambda b, r: (b, r, 0)),
        ],
        out_specs=[
            pl.BlockSpec((1, RB, 9), lambda b, r: (b, r, 0)),
            pl.BlockSpec((1, RB, 3), lambda b, r: (b, r, 0)),
        ],
        out_shape=[
            jax.ShapeDtypeStruct((4, V, 9), jnp.float32),
            jax.ShapeDtypeStruct((4, V, 3), jnp.int32),
        ],
        interpret=_IP,
    )(vt, vertices)
    return nidx, ndn


def _nearest_body(st_ref, t_ref, out_ref, *, W, RB):
    b = pl.program_id(0)
    st = [st_ref[0, d:d + 1, :] for d in range(3)]
    s2 = st[0] * st[0] + st[1] * st[1] + st[2] * st[2]
    tb = [t_ref[0, :, d:d + 1] for d in range(3)]
    t2 = tb[0] * tb[0] + tb[1] * tb[1] + tb[2] * tb[2]
    stb = [_bf(x) for x in st]
    tbb = [_bf(x) for x in tb]
    inner = (tbb[0] * stb[0] + tbb[1] * stb[1]) + tbb[2] * stb[2]
    d = (s2 + t2) - 2.0 * inner
    colI = lax.broadcasted_iota(jnp.int32, (RB, W), 1)
    m = jnp.min(d, axis=1, keepdims=True)
    a = jnp.min(jnp.where(d == m, colI, jnp.int32(2 ** 30)),
                axis=1, keepdims=True)
    out_ref[0, :, :] = a + b * W


def _nearest(target, source, RB=512):
    _, V, _ = target.shape
    W = source.shape[1]
    st = jnp.transpose(source, (0, 2, 1))
    return pl.pallas_call(
        functools.partial(_nearest_body, W=W, RB=RB),
        grid=(4, V // RB),
        in_specs=[
            pl.BlockSpec((1, 3, W), lambda b, r: (b, 0, 0)),
            pl.BlockSpec((1, RB, 3), lambda b, r: (b, r, 0)),
        ],
        out_specs=pl.BlockSpec((1, RB, 1), lambda b, r: (b, r, 0)),
        out_shape=jax.ShapeDtypeStruct((4, V, 1), jnp.int32),
        interpret=_IP,
    )(st, target)


def _surface_mm_body(ndn_ref, sdn_ref, w_ref, b_ref, fm0_ref, fc_ref,
                     fos_ref, *, C):
    sdnn = _sdn_norm(sdn_ref[...])
    mx = None
    for n in range(3):
        th = jax.nn.relu(jnp.dot(ndn_ref[:, 3 * n:3 * n + 3], sdnn,
                                 preferred_element_type=jnp.float32))
        mx = th if mx is None else jnp.maximum(mx, th)
    acc = mx[:, 0:C]
    for s in range(1, _S):
        acc = acc + mx[:, s * C:(s + 1) * C]
    fm0 = jax.nn.relu(acc)
    fm0_ref[...] = fm0
    y = jnp.dot(fm0, w_ref[...], preferred_element_type=jnp.float32) \
        + b_ref[...]
    fc_ref[...] = y[:, :C]
    fos_ref[...] = y[:, C:]


def _surface_mm(ndn, sdn, w, b, C, RB=256):
    N = ndn.shape[0]
    OUT = w.shape[1]
    return pl.pallas_call(
        functools.partial(_surface_mm_body, C=C),
        grid=(N // RB,),
        in_specs=[
            pl.BlockSpec((RB, 9), lambda r: (r, 0)),
            pl.BlockSpec((3, _S * C), lambda r: (0, 0)),
            pl.BlockSpec((w.shape[0], OUT), lambda r: (0, 0)),
            pl.BlockSpec((1, OUT), lambda r: (0, 0)),
        ],
        out_specs=[
            pl.BlockSpec((RB, C), lambda r: (r, 0)),
            pl.BlockSpec((RB, C), lambda r: (r, 0)),
            pl.BlockSpec((RB, OUT - C), lambda r: (r, 0)),
        ],
        out_shape=[
            jax.ShapeDtypeStruct((N, C), jnp.float32),
            jax.ShapeDtypeStruct((N, C), jnp.float32),
            jax.ShapeDtypeStruct((N, OUT - C), jnp.float32),
        ],
        interpret=_IP,
    )(ndn, sdn, w, b.reshape(1, OUT))


def _pool_mm_body(g0_ref, g1_ref, sc_ref, sh_ref, w_ref, b_ref, fc_ref,
                  fos_ref, *, C):
    x0 = jax.nn.relu(g0_ref[...] * sc_ref[...] + sh_ref[...])
    x1 = jax.nn.relu(g1_ref[...] * sc_ref[...] + sh_ref[...])
    x = jnp.maximum(x0, x1)
    y = jnp.dot(x, w_ref[...], preferred_element_type=jnp.float32) \
        + b_ref[...]
    fc_ref[...] = y[:, :C]
    fos_ref[...] = y[:, C:]


def _pool_mm(g, scale, shift, w, b, C, RB=256):
    N2, K = g.shape
    N = N2 // 2
    NBLK = N // RB
    OUT = w.shape[1]
    return pl.pallas_call(
        functools.partial(_pool_mm_body, C=C),
        grid=(NBLK,),
        in_specs=[
            pl.BlockSpec((RB, K), lambda r: (r, 0)),
            pl.BlockSpec((RB, K), lambda r: (r + NBLK, 0)),
            pl.BlockSpec((1, K), lambda r: (0, 0)),
            pl.BlockSpec((1, K), lambda r: (0, 0)),
            pl.BlockSpec((K, OUT), lambda r: (0, 0)),
            pl.BlockSpec((1, OUT), lambda r: (0, 0)),
        ],
        out_specs=[
            pl.BlockSpec((RB, C), lambda r: (r, 0)),
            pl.BlockSpec((RB, OUT - C), lambda r: (r, 0)),
        ],
        out_shape=[
            jax.ShapeDtypeStruct((N, C), jnp.float32),
            jax.ShapeDtypeStruct((N, OUT - C), jnp.float32),
        ],
        interpret=_IP,
    )(g, g, scale.reshape(1, K), shift.reshape(1, K), w, b.reshape(1, OUT))


def _affine_mm_body(x_ref, sc_ref, sh_ref, w_ref, b_ref, fc_ref, fos_ref,
                    *, C):
    x = jax.nn.relu(x_ref[...] * sc_ref[...] + sh_ref[...])
    y = jnp.dot(x, w_ref[...], preferred_element_type=jnp.float32) \
        + b_ref[...]
    fc_ref[...] = y[:, :C]
    fos_ref[...] = y[:, C:]


def _affine_mm(x, scale, shift, w, b, C, RB=256):
    N, K = x.shape
    OUT = w.shape[1]
    return pl.pallas_call(
        functools.partial(_affine_mm_body, C=C),
        grid=(N // RB,),
        in_specs=[
            pl.BlockSpec((RB, K), lambda r: (r, 0)),
            pl.BlockSpec((1, K), lambda r: (0, 0)),
            pl.BlockSpec((1, K), lambda r: (0, 0)),
            pl.BlockSpec((K, OUT), lambda r: (0, 0)),
            pl.BlockSpec((1, OUT), lambda r: (0, 0)),
        ],
        out_specs=[
            pl.BlockSpec((RB, C), lambda r: (r, 0)),
            pl.BlockSpec((RB, OUT - C), lambda r: (r, 0)),
        ],
        out_shape=[
            jax.ShapeDtypeStruct((N, C), jnp.float32),
            jax.ShapeDtypeStruct((N, OUT - C), jnp.float32),
        ],
        interpret=_IP,
    )(x, scale.reshape(1, K), shift.reshape(1, K), w, b.reshape(1, OUT))


def _sdn_norm(sdn):
    nrm = jnp.sqrt(jnp.sum(sdn * sdn, axis=0, keepdims=True))
    return sdn / jnp.maximum(nrm, 1e-12)


def _combine_body(ndn_ref, sdn_ref, fc_ref, f0_ref, f1_ref, f2_ref,
                  out_ref, st_ref, acc_ref, *, C, RB, NBLK):
    r = pl.program_id(0)
    sdnn = _sdn_norm(sdn_ref[...])
    fsr = [f0_ref, f1_ref, f2_ref]
    mx = None
    for n in range(3):
        th = jax.nn.relu(jnp.dot(ndn_ref[:, 3 * n:3 * n + 3], sdnn,
                                 preferred_element_type=jnp.float32))
        prod = th * fsr[n][...]
        mx = prod if mx is None else jnp.maximum(mx, prod)
    acc = fc_ref[...]
    for s in range(_S):
        acc = acc + mx[:, s * C:(s + 1) * C]
    out_ref[...] = acc

    @pl.when(r == 0)
    def _():
        acc_ref[...] = jnp.zeros_like(acc_ref)
    acc_ref[0:1, :] += jnp.sum(acc, axis=0, keepdims=True)
    acc_ref[1:2, :] += jnp.sum(acc * acc, axis=0, keepdims=True)

    @pl.when(r == NBLK - 1)
    def _():
        st_ref[...] = acc_ref[...]


def _conv_combine(ndn, sdn, fc, fs, C, RB=128):
    N = ndn.shape[0]
    NBLK = N // RB

    def _fmap(n):
        return lambda r: (r + n * NBLK, 0)

    return pl.pallas_call(
        functools.partial(_combine_body, C=C, RB=RB, NBLK=NBLK),
        grid=(NBLK,),
        in_specs=[
            pl.BlockSpec((RB, 9), lambda r: (r, 0)),
            pl.BlockSpec((3, _S * C), lambda r: (0, 0)),
            pl.BlockSpec((RB, C), lambda r: (r, 0)),
            pl.BlockSpec((RB, _S * C), _fmap(0)),
            pl.BlockSpec((RB, _S * C), _fmap(1)),
            pl.BlockSpec((RB, _S * C), _fmap(2)),
        ],
        out_specs=[
            pl.BlockSpec((RB, C), lambda r: (r, 0)),
            pl.BlockSpec((2, C), lambda r: (0, 0)),
        ],
        out_shape=[
            jax.ShapeDtypeStruct((N, C), jnp.float32),
            jax.ShapeDtypeStruct((2, C), jnp.float32),
        ],
        scratch_shapes=[pltpu.VMEM((2, C), jnp.float32)],
        interpret=_IP,
    )(ndn, sdn, fc, fs, fs, fs)


def _bn_scale_shift(stats, n_rows, gamma, beta, eps=1e-5):
    mean = stats[0] / n_rows
    var = stats[1] / n_rows - mean * mean
    scale = gamma / jnp.sqrt(var + eps)
    shift = beta - mean * scale
    return scale, shift


def _assemble_body(fm0_ref, f1_ref, f2_ref, f3_ref, s1_ref, h1_ref,
                   s2_ref, h2_ref, s3_ref, h3_ref, out_ref):
    out_ref[:, 0:128] = fm0_ref[...]
    out_ref[:, 128:256] = jax.nn.relu(f1_ref[...] * s1_ref[...]
                                      + h1_ref[...])
    out_ref[:, 256:512] = jax.nn.relu(f2_ref[...] * s2_ref[...]
                                      + h2_ref[...])
    out_ref[:, 512:768] = jax.nn.relu(f3_ref[...] * s3_ref[...]
                                      + h3_ref[...])


def _assemble(fm0, f1, up, s1, h1, s2, h2, s3, h3, RB=512):
    N = fm0.shape[0]
    NBLK = N // RB
    return pl.pallas_call(
        _assemble_body,
        grid=(NBLK,),
        in_specs=[
            pl.BlockSpec((RB, 128), lambda r: (r, 0)),
            pl.BlockSpec((RB, 128), lambda r: (r, 0)),
            pl.BlockSpec((RB, 256), lambda r: (r, 0)),
            pl.BlockSpec((RB, 256), lambda r: (r + NBLK, 0)),
            pl.BlockSpec((1, 128), lambda r: (0, 0)),
            pl.BlockSpec((1, 128), lambda r: (0, 0)),
            pl.BlockSpec((1, 256), lambda r: (0, 0)),
            pl.BlockSpec((1, 256), lambda r: (0, 0)),
            pl.BlockSpec((1, 256), lambda r: (0, 0)),
            pl.BlockSpec((1, 256), lambda r: (0, 0)),
        ],
        out_specs=pl.BlockSpec((RB, 768), lambda r: (r, 0)),
        out_shape=jax.ShapeDtypeStruct((N, 768), jnp.float32),
        interpret=_IP,
    )(fm0, f1, up, up, s1.reshape(1, -1), h1.reshape(1, -1),
      s2.reshape(1, -1), h2.reshape(1, -1), s3.reshape(1, -1),
      h3.reshape(1, -1))


_NC, _NS = 2, 16
_NW = _NC * _NS


@functools.lru_cache(maxsize=None)
def _make_sc_gather(N, D, M):
    assert M % _NW == 0
    b_per_w = M // _NW
    max_rows = max(8, (120 * 1024) // (D * 4))
    ch = 8
    for c in range(8, max_rows + 1, 8):
        if b_per_w % c == 0:
            ch = c
    nch = b_per_w // ch
    mesh = plsc.VectorSubcoreMesh(core_axis_name="c", subcore_axis_name="s",
                                  num_cores=_NC, num_subcores=_NS)

    @functools.partial(
        pl.kernel,
        out_type=jax.ShapeDtypeStruct((M, D), jnp.float32),
        mesh=mesh,
        scratch_types=[
            pltpu.VMEM((b_per_w,), jnp.int32),
            pltpu.VMEM((ch, D), jnp.float32),
            pltpu.VMEM((ch, D), jnp.float32),
            pltpu.SemaphoreType.DMA,
            pltpu.SemaphoreType.DMA,
        ],
    )
    def k(table_hbm, idx_hbm, out_hbm, idx_v, buf0, buf1, sem0, sem1):
        wid = lax.axis_index("s") * _NC + lax.axis_index("c")
        base = wid * b_per_w
        pltpu.sync_copy(idx_hbm.at[pl.ds(base, b_per_w)], idx_v)
        bufs = [buf0, buf1]
        sems = [sem0, sem1]
        h = [None, None]
        h[0] = pltpu.async_copy(table_hbm.at[idx_v.at[pl.ds(0, ch)]],
                                buf0, sem0)
        for c in range(nch):
            if c + 1 < nch:
                h[(c + 1) % 2] = pltpu.async_copy(
                    table_hbm.at[idx_v.at[pl.ds((c + 1) * ch, ch)]],
                    bufs[(c + 1) % 2], sems[(c + 1) % 2])
            h[c % 2].wait()
            pltpu.sync_copy(bufs[c % 2],
                            out_hbm.at[pl.ds(base + c * ch, ch)])

    return k


def _gather_rows(table, idx):
    N, D = table.shape
    return _make_sc_gather(N, D, idx.shape[0])(table, idx)


def kernel(vertices, dirs0, w1, b1, dirs1, w2, b2, dirs2, w3, b3, dirs3,
           g1, be1, g2, be2, g3, be3):
    bs, V, _ = vertices.shape
    W = V // 2
    N1, N2 = bs * V, bs * W

    nidx, ndn = _knn(vertices, V)
    ndn_f = ndn.reshape(N1, 9)
    v1 = vertices[:, ::2, :]

    fm0, fc1, fo1s = _surface_mm(ndn_f, dirs0, w1, b1, 128)

    nidx_nm = jnp.transpose(nidx, (2, 0, 1)).reshape(-1)
    pool_nm = jnp.transpose(nidx[:, ::2, :2], (2, 0, 1)).reshape(-1)

    fs1 = _gather_rows(fo1s, nidx_nm)
    nidx2, ndn2 = _knn(v1, W)
    ndn2_f = ndn2.reshape(N2, 9)
    nearest = _nearest(vertices, v1).reshape(-1)
    nidx2_nm = jnp.transpose(nidx2, (2, 0, 1)).reshape(-1)
    fm1p, st1 = _conv_combine(ndn_f, dirs1, fc1, fs1, 128)
    sc1, sh1 = _bn_scale_shift(st1, N1, g1, be1)

    pooled = _gather_rows(fm1p, pool_nm)
    fc2, fo2s = _pool_mm(pooled, sc1, sh1, w2, b2, 256)

    fs2 = _gather_rows(fo2s, nidx2_nm)
    fm2p, st2 = _conv_combine(ndn2_f, dirs2, fc2, fs2, 256)
    sc2, sh2 = _bn_scale_shift(st2, N2, g2, be2)

    fc3, fo3s = _affine_mm(fm2p, sc2, sh2, w3, b3, 256)
    fs3 = _gather_rows(fo3s, nidx2_nm)
    fm3p, st3 = _conv_combine(ndn2_f, dirs3, fc3, fs3, 256)
    sc3, sh3 = _bn_scale_shift(st3, N2, g3, be3)

    table = jnp.concatenate([fm2p, fm3p], axis=0)
    up_idx = jnp.concatenate([nearest, nearest + N2])
    up = _gather_rows(table, up_idx)

    out = _assemble(fm0, fm1p, up, sc1, sh1, sc2, sh2, sc3, sh3)
    return out.reshape(bs, V, 768)


# speedup vs baseline: 1.0517x; 1.0517x over previous
import functools

import jax
import jax.numpy as jnp
from jax import lax
from jax.experimental import pallas as pl
from jax.experimental.pallas import tpu as pltpu
from jax.experimental.pallas import tpu_sc as plsc

_S = 7
_INF = float("inf")
_IP = False


def _bf(x):
    return x.astype(jnp.bfloat16).astype(jnp.float32)


def _knn_body(vt_ref, v_ref, ndn_ref, nidx_ref, *, V, RB):
    b = pl.program_id(0)
    vt = [vt_ref[0, d:d + 1, :] for d in range(3)]
    qc = vt[0] * vt[0] + vt[1] * vt[1] + vt[2] * vt[2]
    vb = [v_ref[0, :, d:d + 1] for d in range(3)]
    vtb = [_bf(x) for x in vt]
    vbb = [_bf(x) for x in vb]
    inner = (vbb[0] * vtb[0] + vbb[1] * vtb[1]) + vbb[2] * vtb[2]
    qr = vb[0] * vb[0] + vb[1] * vb[1] + vb[2] * vb[2]
    cur = (-2.0 * inner + qc) + qr
    colI = lax.broadcasted_iota(jnp.int32, (RB, V), 1)
    for n in range(4):
        m = jnp.min(cur, axis=1, keepdims=True)
        a = jnp.min(jnp.where(cur == m, colI, jnp.int32(2 ** 30)),
                    axis=1, keepdims=True)
        oh = colI == a
        cur = jnp.where(oh, _INF, cur)
        if n == 0:
            continue
        nb = [jnp.sum(jnp.where(oh, vt[d], 0.0), axis=1, keepdims=True)
              for d in range(3)]
        dirs = [nb[d] - vb[d] for d in range(3)]
        nrm = jnp.sqrt(dirs[0] * dirs[0] + dirs[1] * dirs[1]
                       + dirs[2] * dirs[2])
        den = jnp.maximum(nrm, 1e-12)
        j = n - 1
        for d in range(3):
            ndn_ref[0, :, 3 * j + d:3 * j + d + 1] = dirs[d] / den
        nidx_ref[0, :, j:j + 1] = a + b * V


def _knn(vertices, V, RB=512):
    vt = jnp.transpose(vertices, (0, 2, 1))
    grid = (4, V // RB)
    ndn, nidx = pl.pallas_call(
        functools.partial(_knn_body, V=V, RB=RB),
        grid=grid,
        in_specs=[
            pl.BlockSpec((1, 3, V), lambda b, r: (b, 0, 0)),
            pl.BlockSpec((1, RB, 3), lambda b, r: (b, r, 0)),
        ],
        out_specs=[
            pl.BlockSpec((1, RB, 9), lambda b, r: (b, r, 0)),
            pl.BlockSpec((1, RB, 3), lambda b, r: (b, r, 0)),
        ],
        out_shape=[
            jax.ShapeDtypeStruct((4, V, 9), jnp.float32),
            jax.ShapeDtypeStruct((4, V, 3), jnp.int32),
        ],
        interpret=_IP,
    )(vt, vertices)
    return nidx, ndn


def _nearest_body(st_ref, t_ref, out_ref, *, W, RB):
    b = pl.program_id(0)
    st = [st_ref[0, d:d + 1, :] for d in range(3)]
    s2 = st[0] * st[0] + st[1] * st[1] + st[2] * st[2]
    tb = [t_ref[0, :, d:d + 1] for d in range(3)]
    t2 = tb[0] * tb[0] + tb[1] * tb[1] + tb[2] * tb[2]
    stb = [_bf(x) for x in st]
    tbb = [_bf(x) for x in tb]
    inner = (tbb[0] * stb[0] + tbb[1] * stb[1]) + tbb[2] * stb[2]
    d = (s2 + t2) - 2.0 * inner
    colI = lax.broadcasted_iota(jnp.int32, (RB, W), 1)
    m = jnp.min(d, axis=1, keepdims=True)
    a = jnp.min(jnp.where(d == m, colI, jnp.int32(2 ** 30)),
                axis=1, keepdims=True)
    out_ref[0, :, :] = a + b * W


def _nearest(target, source, RB=512):
    _, V, _ = target.shape
    W = source.shape[1]
    st = jnp.transpose(source, (0, 2, 1))
    return pl.pallas_call(
        functools.partial(_nearest_body, W=W, RB=RB),
        grid=(4, V // RB),
        in_specs=[
            pl.BlockSpec((1, 3, W), lambda b, r: (b, 0, 0)),
            pl.BlockSpec((1, RB, 3), lambda b, r: (b, r, 0)),
        ],
        out_specs=pl.BlockSpec((1, RB, 1), lambda b, r: (b, r, 0)),
        out_shape=jax.ShapeDtypeStruct((4, V, 1), jnp.int32),
        interpret=_IP,
    )(st, target)


def _surface_mm_body(ndn_ref, sdn_ref, w_ref, b_ref, fm0_ref, fc_ref,
                     fos_ref, *, C):
    sdnn = _sdn_norm(sdn_ref[...])
    mx = None
    for n in range(3):
        th = jax.nn.relu(jnp.dot(ndn_ref[:, 3 * n:3 * n + 3], sdnn,
                                 preferred_element_type=jnp.float32))
        mx = th if mx is None else jnp.maximum(mx, th)
    acc = mx[:, 0:C]
    for s in range(1, _S):
        acc = acc + mx[:, s * C:(s + 1) * C]
    fm0 = jax.nn.relu(acc)
    fm0_ref[...] = fm0
    y = jnp.dot(fm0, w_ref[...], preferred_element_type=jnp.float32) \
        + b_ref[...]
    fc_ref[...] = y[:, :C]
    fos_ref[...] = y[:, C:]


def _surface_mm(ndn, sdn, w, b, C, RB=256):
    N = ndn.shape[0]
    OUT = w.shape[1]
    return pl.pallas_call(
        functools.partial(_surface_mm_body, C=C),
        grid=(N // RB,),
        in_specs=[
            pl.BlockSpec((RB, 9), lambda r: (r, 0)),
            pl.BlockSpec((3, _S * C), lambda r: (0, 0)),
            pl.BlockSpec((w.shape[0], OUT), lambda r: (0, 0)),
            pl.BlockSpec((1, OUT), lambda r: (0, 0)),
        ],
        out_specs=[
            pl.BlockSpec((RB, C), lambda r: (r, 0)),
            pl.BlockSpec((RB, C), lambda r: (r, 0)),
            pl.BlockSpec((RB, OUT - C), lambda r: (r, 0)),
        ],
        out_shape=[
            jax.ShapeDtypeStruct((N, C), jnp.float32),
            jax.ShapeDtypeStruct((N, C), jnp.float32),
            jax.ShapeDtypeStruct((N, OUT - C), jnp.float32),
        ],
        interpret=_IP,
    )(ndn, sdn, w, b.reshape(1, OUT))


def _pool_mm_body(g0_ref, g1_ref, sc_ref, sh_ref, w_ref, b_ref, fc_ref,
                  fos_ref, *, C):
    x0 = jax.nn.relu(g0_ref[...] * sc_ref[...] + sh_ref[...])
    x1 = jax.nn.relu(g1_ref[...] * sc_ref[...] + sh_ref[...])
    x = jnp.maximum(x0, x1)
    y = jnp.dot(x, w_ref[...], preferred_element_type=jnp.float32) \
        + b_ref[...]
    fc_ref[...] = y[:, :C]
    fos_ref[...] = y[:, C:]


def _pool_mm(g, scale, shift, w, b, C, RB=256):
    N2, K = g.shape
    N = N2 // 2
    NBLK = N // RB
    OUT = w.shape[1]
    return pl.pallas_call(
        functools.partial(_pool_mm_body, C=C),
        grid=(NBLK,),
        in_specs=[
            pl.BlockSpec((RB, K), lambda r: (r, 0)),
            pl.BlockSpec((RB, K), lambda r: (r + NBLK, 0)),
            pl.BlockSpec((1, K), lambda r: (0, 0)),
            pl.BlockSpec((1, K), lambda r: (0, 0)),
            pl.BlockSpec((K, OUT), lambda r: (0, 0)),
            pl.BlockSpec((1, OUT), lambda r: (0, 0)),
        ],
        out_specs=[
            pl.BlockSpec((RB, C), lambda r: (r, 0)),
            pl.BlockSpec((RB, OUT - C), lambda r: (r, 0)),
        ],
        out_shape=[
            jax.ShapeDtypeStruct((N, C), jnp.float32),
            jax.ShapeDtypeStruct((N, OUT - C), jnp.float32),
        ],
        interpret=_IP,
    )(g, g, scale.reshape(1, K), shift.reshape(1, K), w, b.reshape(1, OUT))


def _affine_mm_body(x_ref, sc_ref, sh_ref, w_ref, b_ref, fc_ref, fos_ref,
                    *, C):
    x = jax.nn.relu(x_ref[...] * sc_ref[...] + sh_ref[...])
    y = jnp.dot(x, w_ref[...], preferred_element_type=jnp.float32) \
        + b_ref[...]
    fc_ref[...] = y[:, :C]
    fos_ref[...] = y[:, C:]


def _affine_mm(x, scale, shift, w, b, C, RB=256):
    N, K = x.shape
    OUT = w.shape[1]
    return pl.pallas_call(
        functools.partial(_affine_mm_body, C=C),
        grid=(N // RB,),
        in_specs=[
            pl.BlockSpec((RB, K), lambda r: (r, 0)),
            pl.BlockSpec((1, K), lambda r: (0, 0)),
            pl.BlockSpec((1, K), lambda r: (0, 0)),
            pl.BlockSpec((K, OUT), lambda r: (0, 0)),
            pl.BlockSpec((1, OUT), lambda r: (0, 0)),
        ],
        out_specs=[
            pl.BlockSpec((RB, C), lambda r: (r, 0)),
            pl.BlockSpec((RB, OUT - C), lambda r: (r, 0)),
        ],
        out_shape=[
            jax.ShapeDtypeStruct((N, C), jnp.float32),
            jax.ShapeDtypeStruct((N, OUT - C), jnp.float32),
        ],
        interpret=_IP,
    )(x, scale.reshape(1, K), shift.reshape(1, K), w, b.reshape(1, OUT))


def _sdn_norm(sdn):
    nrm = jnp.sqrt(jnp.sum(sdn * sdn, axis=0, keepdims=True))
    return sdn / jnp.maximum(nrm, 1e-12)


def _combine_body(ndn_ref, sdn_ref, fc_ref, f0_ref, f1_ref, f2_ref,
                  out_ref, st_ref, acc_ref, *, C, RB, NBLK):
    r = pl.program_id(0)
    sdnn = _sdn_norm(sdn_ref[...])
    fsr = [f0_ref, f1_ref, f2_ref]
    mx = None
    for n in range(3):
        th = jax.nn.relu(jnp.dot(ndn_ref[:, 3 * n:3 * n + 3], sdnn,
                                 preferred_element_type=jnp.float32))
        prod = th * fsr[n][...]
        mx = prod if mx is None else jnp.maximum(mx, prod)
    acc = fc_ref[...]
    for s in range(_S):
        acc = acc + mx[:, s * C:(s + 1) * C]
    out_ref[...] = acc

    @pl.when(r == 0)
    def _():
        acc_ref[...] = jnp.zeros_like(acc_ref)
    acc_ref[0:1, :] += jnp.sum(acc, axis=0, keepdims=True)
    acc_ref[1:2, :] += jnp.sum(acc * acc, axis=0, keepdims=True)

    @pl.when(r == NBLK - 1)
    def _():
        st_ref[...] = acc_ref[...]


def _conv_combine(ndn, sdn, fc, fs, C, RB=256):
    N = ndn.shape[0]
    NBLK = N // RB

    def _fmap(n):
        return lambda r: (r + n * NBLK, 0)

    return pl.pallas_call(
        functools.partial(_combine_body, C=C, RB=RB, NBLK=NBLK),
        grid=(NBLK,),
        in_specs=[
            pl.BlockSpec((RB, 9), lambda r: (r, 0)),
            pl.BlockSpec((3, _S * C), lambda r: (0, 0)),
            pl.BlockSpec((RB, C), lambda r: (r, 0)),
            pl.BlockSpec((RB, _S * C), _fmap(0)),
            pl.BlockSpec((RB, _S * C), _fmap(1)),
            pl.BlockSpec((RB, _S * C), _fmap(2)),
        ],
        out_specs=[
            pl.BlockSpec((RB, C), lambda r: (r, 0)),
            pl.BlockSpec((2, C), lambda r: (0, 0)),
        ],
        out_shape=[
            jax.ShapeDtypeStruct((N, C), jnp.float32),
            jax.ShapeDtypeStruct((2, C), jnp.float32),
        ],
        scratch_shapes=[pltpu.VMEM((2, C), jnp.float32)],
        interpret=_IP,
    )(ndn, sdn, fc, fs, fs, fs)


def _bn_scale_shift(stats, n_rows, gamma, beta, eps=1e-5):
    mean = stats[0] / n_rows
    var = stats[1] / n_rows - mean * mean
    scale = gamma / jnp.sqrt(var + eps)
    shift = beta - mean * scale
    return scale, shift


def _assemble_body(fm0_ref, f1_ref, f2_ref, f3_ref, s1_ref, h1_ref,
                   s2_ref, h2_ref, s3_ref, h3_ref, out_ref):
    out_ref[:, 0:128] = fm0_ref[...]
    out_ref[:, 128:256] = jax.nn.relu(f1_ref[...] * s1_ref[...]
                                      + h1_ref[...])
    out_ref[:, 256:512] = jax.nn.relu(f2_ref[...] * s2_ref[...]
                                      + h2_ref[...])
    out_ref[:, 512:768] = jax.nn.relu(f3_ref[...] * s3_ref[...]
                                      + h3_ref[...])


def _assemble(fm0, f1, up, s1, h1, s2, h2, s3, h3, RB=512):
    N = fm0.shape[0]
    NBLK = N // RB
    return pl.pallas_call(
        _assemble_body,
        grid=(NBLK,),
        in_specs=[
            pl.BlockSpec((RB, 128), lambda r: (r, 0)),
            pl.BlockSpec((RB, 128), lambda r: (r, 0)),
            pl.BlockSpec((RB, 256), lambda r: (r, 0)),
            pl.BlockSpec((RB, 256), lambda r: (r + NBLK, 0)),
            pl.BlockSpec((1, 128), lambda r: (0, 0)),
            pl.BlockSpec((1, 128), lambda r: (0, 0)),
            pl.BlockSpec((1, 256), lambda r: (0, 0)),
            pl.BlockSpec((1, 256), lambda r: (0, 0)),
            pl.BlockSpec((1, 256), lambda r: (0, 0)),
            pl.BlockSpec((1, 256), lambda r: (0, 0)),
        ],
        out_specs=pl.BlockSpec((RB, 768), lambda r: (r, 0)),
        out_shape=jax.ShapeDtypeStruct((N, 768), jnp.float32),
        interpret=_IP,
    )(fm0, f1, up, up, s1.reshape(1, -1), h1.reshape(1, -1),
      s2.reshape(1, -1), h2.reshape(1, -1), s3.reshape(1, -1),
      h3.reshape(1, -1))


_NC, _NS = 2, 16
_NW = _NC * _NS


@functools.lru_cache(maxsize=None)
def _make_sc_gather(N, D, M):
    assert M % _NW == 0
    b_per_w = M // _NW
    max_rows = max(8, (120 * 1024) // (D * 4))
    ch = 8
    for c in range(8, max_rows + 1, 8):
        if b_per_w % c == 0:
            ch = c
    nch = b_per_w // ch
    mesh = plsc.VectorSubcoreMesh(core_axis_name="c", subcore_axis_name="s",
                                  num_cores=_NC, num_subcores=_NS)

    @functools.partial(
        pl.kernel,
        out_type=jax.ShapeDtypeStruct((M, D), jnp.float32),
        mesh=mesh,
        scratch_types=[
            pltpu.VMEM((b_per_w,), jnp.int32),
            pltpu.VMEM((ch, D), jnp.float32),
            pltpu.VMEM((ch, D), jnp.float32),
            pltpu.SemaphoreType.DMA,
            pltpu.SemaphoreType.DMA,
        ],
    )
    def k(table_hbm, idx_hbm, out_hbm, idx_v, buf0, buf1, sem0, sem1):
        wid = lax.axis_index("s") * _NC + lax.axis_index("c")
        base = wid * b_per_w
        pltpu.sync_copy(idx_hbm.at[pl.ds(base, b_per_w)], idx_v)
        bufs = [buf0, buf1]
        sems = [sem0, sem1]
        h = [None, None]
        h[0] = pltpu.async_copy(table_hbm.at[idx_v.at[pl.ds(0, ch)]],
                                buf0, sem0)
        for c in range(nch):
            if c + 1 < nch:
                h[(c + 1) % 2] = pltpu.async_copy(
                    table_hbm.at[idx_v.at[pl.ds((c + 1) * ch, ch)]],
                    bufs[(c + 1) % 2], sems[(c + 1) % 2])
            h[c % 2].wait()
            pltpu.sync_copy(bufs[c % 2],
                            out_hbm.at[pl.ds(base + c * ch, ch)])

    return k


def _gather_rows(table, idx):
    N, D = table.shape
    return _make_sc_gather(N, D, idx.shape[0])(table, idx)


def kernel(vertices, dirs0, w1, b1, dirs1, w2, b2, dirs2, w3, b3, dirs3,
           g1, be1, g2, be2, g3, be3):
    bs, V, _ = vertices.shape
    W = V // 2
    N1, N2 = bs * V, bs * W

    nidx, ndn = _knn(vertices, V)
    ndn_f = ndn.reshape(N1, 9)
    v1 = vertices[:, ::2, :]

    fm0, fc1, fo1s = _surface_mm(ndn_f, dirs0, w1, b1, 128)

    nidx_nm = jnp.transpose(nidx, (2, 0, 1)).reshape(-1)
    pool_nm = jnp.transpose(nidx[:, ::2, :2], (2, 0, 1)).reshape(-1)

    fs1 = _gather_rows(fo1s, nidx_nm)
    nidx2, ndn2 = _knn(v1, W)
    ndn2_f = ndn2.reshape(N2, 9)
    nearest = _nearest(vertices, v1).reshape(-1)
    nidx2_nm = jnp.transpose(nidx2, (2, 0, 1)).reshape(-1)
    fm1p, st1 = _conv_combine(ndn_f, dirs1, fc1, fs1, 128)
    sc1, sh1 = _bn_scale_shift(st1, N1, g1, be1)

    pooled = _gather_rows(fm1p, pool_nm)
    fc2, fo2s = _pool_mm(pooled, sc1, sh1, w2, b2, 256)

    fs2 = _gather_rows(fo2s, nidx2_nm)
    fm2p, st2 = _conv_combine(ndn2_f, dirs2, fc2, fs2, 256)
    sc2, sh2 = _bn_scale_shift(st2, N2, g2, be2)

    fc3, fo3s = _affine_mm(fm2p, sc2, sh2, w3, b3, 256)
    fs3 = _gather_rows(fo3s, nidx2_nm)
    fm3p, st3 = _conv_combine(ndn2_f, dirs3, fc3, fs3, 256)
    sc3, sh3 = _bn_scale_shift(st3, N2, g3, be3)

    table = jnp.concatenate([fm2p, fm3p], axis=0)
    up_idx = jnp.concatenate([nearest, nearest + N2])
    up = _gather_rows(table, up_idx)

    out = _assemble(fm0, fm1p, up, sc1, sh1, sc2, sh2, sc3, sh3)
    return out.reshape(bs, V, 768)
